# Initial kernel scaffold; baseline (speedup 1.0000x reference)
import functools

import jax
import jax.numpy as jnp
from jax import lax
from jax.experimental import pallas as pl
from jax.experimental.pallas import tpu as pltpu
from jax.experimental.pallas import tpu_sc as plsc

NC = 2
NS = 16
NW = NC * NS
L = 16


def _sc_scatter(N, D, E, W, T, K):
    EPT = E // NW
    CH = EPT // K
    RPT = N // NS
    ZR = 125
    T_pad = ((T + L - 1) // L) * L
    TG = T_pad // L

    mesh = plsc.VectorSubcoreMesh(core_axis_name="core", subcore_axis_name="subcore")

    @functools.partial(
        pl.kernel,
        mesh=mesh,
        out_type=[
            jax.ShapeDtypeStruct((N, D), jnp.float32),
            jax.ShapeDtypeStruct((N, D), jnp.float32),
            jax.ShapeDtypeStruct((N,), jnp.float32),
        ],
        scratch_types=[
            pltpu.VMEM((CH, K), jnp.int32),
            pltpu.VMEM((CH, K), jnp.int32),
            pltpu.VMEM((EPT,), jnp.int32),
            pltpu.VMEM((EPT,), jnp.float32),
            pltpu.VMEM((W,), jnp.float32),
            pltpu.VMEM((K, D), jnp.float32),
            pltpu.VMEM((125, D), jnp.float32),
            pltpu.VMEM((N,), jnp.float32),
            pltpu.VMEM((((T + 15) // 16) * 16,), jnp.int32),
            pltpu.VMEM_SHARED((N, D), jnp.float32),
        ],
    )
    def k(x_hbm, w_hbm, u_hbm, v_hbm, widx_hbm, tgt_hbm,
          p0_hbm, p1_hbm, mask_hbm,
          u_v, v_v, widx_v, wvals_v, w_v, rows_v, zb_v, mask_v, tgt_v, acc):
        ZRl = 125
        c = lax.axis_index("core")
        s = lax.axis_index("subcore")
        wid = c * NS + s

        pltpu.sync_copy(u_hbm.at[wid], u_v)
        pltpu.sync_copy(v_hbm.at[wid], v_v)
        pltpu.sync_copy(widx_hbm.at[wid], widx_v)
        pltpu.sync_copy(w_hbm, w_v)

        @pl.loop(0, EPT // L)
        def _(g):
            idx16 = widx_v[pl.ds(g * L, L)]
            wvals_v[pl.ds(g * L, L)] = plsc.load_gather(w_v, [idx16])

        @pl.loop(0, ZRl)
        def _(r):
            for j in range(D // L):
                zb_v[r, pl.ds(j * L, L)] = jnp.zeros((L,), jnp.float32)

        for j in range(RPT // ZRl):
            pltpu.sync_copy(zb_v, acc.at[pl.ds(s * RPT + j * ZRl, ZRl)])

        @pl.when(jnp.logical_and(c == 0, s == 0))
        def _():
            pltpu.sync_copy(tgt_hbm, tgt_v.at[pl.ds(0, T)])

            @pl.loop(0, N // L)
            def _(r):
                mask_v[pl.ds(r * L, L)] = jnp.ones((L,), jnp.float32)

            zeros16 = jnp.zeros((L,), jnp.float32)
            iota16 = lax.iota(jnp.int32, L)

            @pl.loop(0, TG)
            def _(g):
                tgt16 = tgt_v[pl.ds(g * L, L)]
                lanemask = (g * L + iota16) < T
                plsc.store_scatter(mask_v, [tgt16], zeros16, mask=lanemask)

            pltpu.sync_copy(mask_v, mask_hbm)

        plsc.subcore_barrier()

        @pl.loop(0, CH)
        def _(i):
            pltpu.sync_copy(x_hbm.at[u_v.at[i]], rows_v)

            @pl.loop(0, K)
            def _(e):
                bc = plsc.load_gather(wvals_v, [jnp.full((L,), i * K + e, jnp.int32)])
                for j in range(D // L):
                    sl = pl.ds(j * L, L)
                    rows_v[e, sl] = rows_v[e, sl] * bc

            pltpu.sync_copy(rows_v, acc.at[v_v.at[i]], add=True)

        plsc.subcore_barrier()

        rows_sl = pl.ds(s * RPT, RPT)

        @pl.when(c == 0)
        def _():
            pltpu.sync_copy(acc.at[rows_sl], p0_hbm.at[rows_sl])

        @pl.when(c == 1)
        def _():
            pltpu.sync_copy(acc.at[rows_sl], p1_hbm.at[rows_sl])

    return k


def _combine_body(x_ref, m_ref, p0_ref, p1_ref, o_ref):
    o_ref[...] = x_ref[...] * m_ref[...] + p0_ref[...] + p1_ref[...]


def kernel(layer_input, weights, u, v, weight_idx, targets):
    N, D = layer_input.shape
    (E,) = u.shape
    (W,) = weights.shape
    (T,) = targets.shape
    K = 80

    u = u.astype(jnp.int32).reshape(NW, (E // NW) // K, K)
    v = v.astype(jnp.int32).reshape(NW, (E // NW) // K, K)
    weight_idx = weight_idx.astype(jnp.int32).reshape(NW, E // NW)
    targets = targets.astype(jnp.int32)

    p0, p1, mask = _sc_scatter(N, D, E, W, T, K)(
        layer_input, weights, u, v, weight_idx, targets)

    BN = 2000
    out = pl.pallas_call(
        _combine_body,
        grid=(N // BN,),
        in_specs=[
            pl.BlockSpec((BN, D), lambda i: (i, 0)),
            pl.BlockSpec((BN, 1), lambda i: (i, 0)),
            pl.BlockSpec((BN, D), lambda i: (i, 0)),
            pl.BlockSpec((BN, D), lambda i: (i, 0)),
        ],
        out_specs=pl.BlockSpec((BN, D), lambda i: (i, 0)),
        out_shape=jax.ShapeDtypeStruct((N, D), jnp.float32),
    )(layer_input, mask.reshape(N, 1), p0, p1)
    return out


# speedup vs baseline: 7.2454x; 7.2454x over previous
import dataclasses
import functools

import jax
import jax.numpy as jnp
from jax import lax
from jax.experimental import pallas as pl
from jax.experimental.pallas import tpu as pltpu
from jax.experimental.pallas import tpu_sc as plsc

NC = 2
NS = 16
L = 16


def _sc_scatter(N, D, E, W, T, K):
    DH = D // 2
    EPT = E // NS
    CH = EPT // K
    A = (N // NS) // 8 * 8
    REM = N - NS * A
    ZR = 104
    T_pad = ((T + L - 1) // L) * L
    TG = T_pad // L

    mesh = plsc.VectorSubcoreMesh(core_axis_name="core", subcore_axis_name="subcore")

    cp = pltpu.CompilerParams()
    if "needs_layout_passes" in pltpu.CompilerParams.__dataclass_fields__:
        cp = dataclasses.replace(cp, needs_layout_passes=False)
    if "use_tc_tiling_on_sc" in pltpu.CompilerParams.__dataclass_fields__:
        cp = dataclasses.replace(cp, use_tc_tiling_on_sc=False)

    @functools.partial(
        pl.kernel,
        mesh=mesh,
        compiler_params=cp,
        out_type=[
            jax.ShapeDtypeStruct((N, DH), jnp.float32),
            jax.ShapeDtypeStruct((N, DH), jnp.float32),
            jax.ShapeDtypeStruct((N,), jnp.float32),
        ],
        scratch_types=[
            pltpu.VMEM((CH, K), jnp.int32),
            pltpu.VMEM((CH, K), jnp.int32),
            pltpu.VMEM((EPT,), jnp.int32),
            pltpu.VMEM((W,), jnp.float32),
            pltpu.VMEM((K, DH), jnp.float32),
            pltpu.VMEM((N,), jnp.float32),
            pltpu.VMEM((((T + 15) // 16) * 16,), jnp.int32),
            pltpu.VMEM_SHARED((N, DH), jnp.float32),
        ],
    )
    def k(xl_hbm, xr_hbm, w_hbm, u_hbm, v_hbm, widx_hbm, tgt_hbm,
          p0_hbm, p1_hbm, mask_hbm,
          u_v, v_v, wval_v, w_v, rows_v, mask_v, tgt_v, acc):
        c = lax.axis_index("core")
        s = lax.axis_index("subcore")

        pltpu.sync_copy(u_hbm.at[s], u_v)
        pltpu.sync_copy(v_hbm.at[s], v_v)
        pltpu.sync_copy(widx_hbm.at[s], wval_v)
        pltpu.sync_copy(w_hbm, w_v)

        @pl.loop(0, EPT // L)
        def _(g):
            idx16 = wval_v[pl.ds(g * L, L)]
            vals = plsc.load_gather(w_v, [idx16])
            wval_v[pl.ds(g * L, L)] = plsc.bitcast(vals, jnp.int32)

        @pl.loop(0, K)
        def _(r):
            for j in range(DH // L):
                rows_v[r, pl.ds(j * L, L)] = jnp.zeros((L,), jnp.float32)

        base = pl.multiple_of(s * A, 8)
        for j in range(A // K):
            pltpu.sync_copy(rows_v, acc.at[pl.ds(base + j * K, K)])
        if A % K:
            pltpu.sync_copy(rows_v.at[pl.ds(0, A % K)],
                            acc.at[pl.ds(base + (A // K) * K, A % K)])

        @pl.when(s < REM // 8)
        def _():
            rbase = pl.multiple_of(NS * A + s * 8, 8)
            pltpu.sync_copy(rows_v.at[pl.ds(0, 8)], acc.at[pl.ds(rbase, 8)])

        @pl.when(jnp.logical_and(c == 0, s == 0))
        def _():
            pltpu.sync_copy(tgt_hbm, tgt_v.at[pl.ds(0, T)])

            @pl.loop(0, N // L)
            def _(r):
                mask_v[pl.ds(r * L, L)] = jnp.ones((L,), jnp.float32)

            zeros16 = jnp.zeros((L,), jnp.float32)
            iota16 = lax.iota(jnp.int32, L)

            @pl.loop(0, TG)
            def _(g):
                tgt16 = tgt_v[pl.ds(g * L, L)]
                lanemask = (g * L + iota16) < T
                plsc.store_scatter(mask_v, [tgt16], zeros16, mask=lanemask)

            pltpu.sync_copy(mask_v, mask_hbm)

        plsc.subcore_barrier()

        def edge_loop(x_src):
            @pl.loop(0, CH)
            def _(i):
                pltpu.sync_copy(x_src.at[u_v.at[i]], rows_v)

                @pl.loop(0, K)
                def _(e):
                    bits = plsc.load_gather(
                        wval_v, [jnp.full((L,), i * K + e, jnp.int32)])
                    bc = plsc.bitcast(bits, jnp.float32)
                    for j in range(DH // L):
                        sl = pl.ds(j * L, L)
                        rows_v[e, sl] = rows_v[e, sl] * bc

                pltpu.sync_copy(rows_v, acc.at[v_v.at[i]], add=True)

        @pl.when(c == 0)
        def _():
            edge_loop(xl_hbm)

        @pl.when(c == 1)
        def _():
            edge_loop(xr_hbm)

        plsc.subcore_barrier()

        rows_sl = pl.ds(base, A)
        rem_sl = pl.ds(pl.multiple_of(NS * A + s * 8, 8), 8)

        @pl.when(c == 0)
        def _():
            pltpu.sync_copy(acc.at[rows_sl], p0_hbm.at[rows_sl])

            @pl.when(s < REM // 8)
            def _():
                pltpu.sync_copy(acc.at[rem_sl], p0_hbm.at[rem_sl])

        @pl.when(c == 1)
        def _():
            pltpu.sync_copy(acc.at[rows_sl], p1_hbm.at[rows_sl])

            @pl.when(s < REM // 8)
            def _():
                pltpu.sync_copy(acc.at[rem_sl], p1_hbm.at[rem_sl])

    return k


def _combine_body(x_ref, m_ref, p0_ref, p1_ref, o_ref):
    agg = jnp.concatenate([p0_ref[...], p1_ref[...]], axis=1)
    o_ref[...] = x_ref[...] * m_ref[...] + agg


def kernel(layer_input, weights, u, v, weight_idx, targets):
    N, D = layer_input.shape
    (E,) = u.shape
    (W,) = weights.shape
    (T,) = targets.shape
    K = 80
    DH = D // 2

    xl = layer_input[:, :DH]
    xr = layer_input[:, DH:]
    u = u.astype(jnp.int32).reshape(NS, (E // NS) // K, K)
    v = v.astype(jnp.int32).reshape(NS, (E // NS) // K, K)
    weight_idx = weight_idx.astype(jnp.int32).reshape(NS, E // NS)
    targets = targets.astype(jnp.int32)

    p0, p1, mask = _sc_scatter(N, D, E, W, T, K)(
        xl, xr, weights, u, v, weight_idx, targets)

    BN = 2000
    out = pl.pallas_call(
        _combine_body,
        grid=(N // BN,),
        in_specs=[
            pl.BlockSpec((BN, D), lambda i: (i, 0)),
            pl.BlockSpec((BN, 1), lambda i: (i, 0)),
            pl.BlockSpec((BN, DH), lambda i: (i, 0)),
            pl.BlockSpec((BN, DH), lambda i: (i, 0)),
        ],
        out_specs=pl.BlockSpec((BN, D), lambda i: (i, 0)),
        out_shape=jax.ShapeDtypeStruct((N, D), jnp.float32),
    )(layer_input, mask.reshape(N, 1), p0, p1)
    return out


# speedup vs baseline: 13.9935x; 1.9314x over previous
import dataclasses
import functools

import jax
import jax.numpy as jnp
from jax import lax
from jax.experimental import pallas as pl
from jax.experimental.pallas import tpu as pltpu
from jax.experimental.pallas import tpu_sc as plsc

NC = 2
NS = 16
L = 16


def _sc_scatter(N, D, E, W, T, K):
    DH = D // 2
    EPT = E // NS
    CH = EPT // K
    A = (N // NS) // 8 * 8
    REM = N - NS * A
    TH0 = ((T + 1) // 2 + 7) // 8 * 8
    TH1 = T - TH0
    FULL = (CH - 1) // 3

    mesh = plsc.VectorSubcoreMesh(core_axis_name="core", subcore_axis_name="subcore")

    cp = pltpu.CompilerParams()
    if "needs_layout_passes" in pltpu.CompilerParams.__dataclass_fields__:
        cp = dataclasses.replace(cp, needs_layout_passes=False)
    if "use_tc_tiling_on_sc" in pltpu.CompilerParams.__dataclass_fields__:
        cp = dataclasses.replace(cp, use_tc_tiling_on_sc=False)

    @functools.partial(
        pl.kernel,
        mesh=mesh,
        compiler_params=cp,
        out_type=[
            jax.ShapeDtypeStruct((N, DH), jnp.float32),
            jax.ShapeDtypeStruct((N, DH), jnp.float32),
            jax.ShapeDtypeStruct((N,), jnp.float32),
        ],
        scratch_types=[
            pltpu.VMEM((CH, K), jnp.int32),
            pltpu.VMEM((CH, K), jnp.int32),
            pltpu.VMEM((EPT,), jnp.int32),
            pltpu.VMEM((W,), jnp.float32),
            pltpu.VMEM((K, DH), jnp.float32),
            pltpu.VMEM((K, DH), jnp.float32),
            pltpu.VMEM((K, DH), jnp.float32),
            pltpu.VMEM((N,), jnp.float32),
            pltpu.VMEM((TH0,), jnp.int32),
            pltpu.VMEM_SHARED((N, DH), jnp.float32),
            pltpu.SemaphoreType.DMA,
            pltpu.SemaphoreType.DMA,
            pltpu.SemaphoreType.DMA,
            pltpu.SemaphoreType.DMA,
            pltpu.SemaphoreType.DMA,
            pltpu.SemaphoreType.DMA,
        ],
    )
    def k(xl_hbm, xr_hbm, w_hbm, u_hbm, v_hbm, widx_hbm, tgt_hbm,
          p0_hbm, p1_hbm, mask_hbm,
          u_v, v_v, wval_v, w_v, rows0_v, rows1_v, rows2_v, mask_v, tgt_v,
          acc, g0, g1, g2, s0, s1, s2):
        c = lax.axis_index("core")
        s = lax.axis_index("subcore")
        bufs = (rows0_v, rows1_v, rows2_v)
        gsems = (g0, g1, g2)
        ssems = (s0, s1, s2)

        pltpu.sync_copy(u_hbm.at[s], u_v)
        pltpu.sync_copy(v_hbm.at[s], v_v)
        pltpu.sync_copy(widx_hbm.at[s], wval_v)
        pltpu.sync_copy(w_hbm, w_v)

        @pl.loop(0, EPT // L)
        def _(g):
            idx16 = wval_v[pl.ds(g * L, L)]
            vals = plsc.load_gather(w_v, [idx16])
            wval_v[pl.ds(g * L, L)] = plsc.bitcast(vals, jnp.int32)

        @pl.loop(0, K)
        def _(r):
            for j in range(DH // L):
                rows0_v[r, pl.ds(j * L, L)] = jnp.zeros((L,), jnp.float32)

        base = pl.multiple_of(s * A, 8)
        for j in range(A // K):
            pltpu.sync_copy(rows0_v, acc.at[pl.ds(base + j * K, K)])
        if A % K:
            pltpu.sync_copy(rows0_v.at[pl.ds(0, A % K)],
                            acc.at[pl.ds(base + (A // K) * K, A % K)])

        @pl.when(s < REM // 8)
        def _():
            rbase = pl.multiple_of(NS * A + s * 8, 8)
            pltpu.sync_copy(rows0_v.at[pl.ds(0, 8)], acc.at[pl.ds(rbase, 8)])

        plsc.subcore_barrier()

        def edge_loop(x_src):
            def g_start(kk, b):
                pltpu.async_copy(x_src.at[u_v.at[kk]], bufs[b], gsems[b])

            def g_wait(b):
                pltpu.make_async_copy(
                    x_src.at[u_v.at[0]], bufs[b], gsems[b]).wait()

            def s_start(kk, b):
                pltpu.async_copy(bufs[b], acc.at[v_v.at[kk]], ssems[b],
                                 add=True)

            def s_wait(b):
                pltpu.make_async_copy(
                    bufs[b], acc.at[v_v.at[0]], ssems[b]).wait()

            def scale(kk, b):
                buf = bufs[b]

                @pl.loop(0, K, step=4)
                def _(e0):
                    for dd in range(4):
                        e = e0 + dd
                        bits = plsc.load_gather(
                            wval_v, [jnp.full((L,), kk * K + e, jnp.int32)])
                        bc = plsc.bitcast(bits, jnp.float32)
                        for j in range(DH // L):
                            sl = pl.ds(j * L, L)
                            buf[e, sl] = buf[e, sl] * bc

            g_start(0, 0)

            @pl.loop(0, FULL)
            def _(i):
                for d in range(3):
                    kk = 3 * i + d
                    b = d

                    @pl.when(kk >= 2)
                    def _():
                        s_wait((b + 1) % 3)

                    g_start(kk + 1, (b + 1) % 3)
                    g_wait(b)
                    scale(kk, b)
                    s_start(kk, b)

            for kk in range(3 * FULL, CH):
                b = kk % 3
                s_wait((b + 1) % 3)
                if kk + 1 < CH:
                    g_start(kk + 1, (b + 1) % 3)
                g_wait(b)
                scale(kk, b)
                s_start(kk, b)

            s_wait((CH - 2) % 3)
            s_wait((CH - 1) % 3)

        @pl.when(c == 0)
        def _():
            edge_loop(xl_hbm)

        @pl.when(c == 1)
        def _():
            edge_loop(xr_hbm)

        plsc.subcore_barrier()

        rows_sl = pl.ds(base, A)
        rem_sl = pl.ds(pl.multiple_of(NS * A + s * 8, 8), 8)

        @pl.when(c == 0)
        def _():
            pltpu.sync_copy(acc.at[rows_sl], p0_hbm.at[rows_sl])

            @pl.when(s < REM // 8)
            def _():
                pltpu.sync_copy(acc.at[rem_sl], p0_hbm.at[rem_sl])

        @pl.when(c == 1)
        def _():
            pltpu.sync_copy(acc.at[rows_sl], p1_hbm.at[rows_sl])

            @pl.when(s < REM // 8)
            def _():
                pltpu.sync_copy(acc.at[rem_sl], p1_hbm.at[rem_sl])

        @pl.when(jnp.logical_and(c == 0, s == 0))
        def _():
            @pl.loop(0, N // L)
            def _(r):
                mask_v[pl.ds(r * L, L)] = jnp.ones((L,), jnp.float32)

            zeros16 = jnp.zeros((L,), jnp.float32)
            iota16 = lax.iota(jnp.int32, L)

            for half, (toff, tlen) in enumerate(((0, TH0), (TH0, TH1))):
                pltpu.sync_copy(tgt_hbm.at[pl.ds(toff, tlen)],
                                tgt_v.at[pl.ds(0, tlen)])
                ngroups = (tlen + L - 1) // L

                @pl.loop(0, ngroups)
                def _(g, tlen=tlen):
                    tgt16 = tgt_v[pl.ds(g * L, L)]
                    lanemask = (g * L + iota16) < tlen
                    plsc.store_scatter(mask_v, [tgt16], zeros16,
                                       mask=lanemask)

            pltpu.sync_copy(mask_v, mask_hbm)

    return k


def _combine_body(x_ref, m_ref, p0_ref, p1_ref, o_ref):
    agg = jnp.concatenate([p0_ref[...], p1_ref[...]], axis=1)
    o_ref[...] = x_ref[...] * m_ref[...] + agg


def kernel(layer_input, weights, u, v, weight_idx, targets):
    N, D = layer_input.shape
    (E,) = u.shape
    (W,) = weights.shape
    (T,) = targets.shape
    K = 80
    DH = D // 2

    xl = layer_input[:, :DH]
    xr = layer_input[:, DH:]
    u = u.astype(jnp.int32).reshape(NS, (E // NS) // K, K)
    v = v.astype(jnp.int32).reshape(NS, (E // NS) // K, K)
    weight_idx = weight_idx.astype(jnp.int32).reshape(NS, E // NS)
    targets = targets.astype(jnp.int32)

    p0, p1, mask = _sc_scatter(N, D, E, W, T, K)(
        xl, xr, weights, u, v, weight_idx, targets)

    BN = 2000
    out = pl.pallas_call(
        _combine_body,
        grid=(N // BN,),
        in_specs=[
            pl.BlockSpec((BN, D), lambda i: (i, 0)),
            pl.BlockSpec((BN, 1), lambda i: (i, 0)),
            pl.BlockSpec((BN, DH), lambda i: (i, 0)),
            pl.BlockSpec((BN, DH), lambda i: (i, 0)),
        ],
        out_specs=pl.BlockSpec((BN, D), lambda i: (i, 0)),
        out_shape=jax.ShapeDtypeStruct((N, D), jnp.float32),
    )(layer_input, mask.reshape(N, 1), p0, p1)
    return out


# speedup vs baseline: 14.3977x; 1.0289x over previous
import dataclasses
import functools

import jax
import jax.numpy as jnp
from jax import lax
from jax.experimental import pallas as pl
from jax.experimental.pallas import tpu as pltpu
from jax.experimental.pallas import tpu_sc as plsc

NC = 2
NS = 16
L = 16
K = 80
BATCH = 10
NRB = 5


def _sc_scatter(N, D, E, W, T):
    DH = D // 2
    EPT = E // NS
    CH = EPT // K
    NB = CH // BATCH
    A = (N // NS) // 8 * 8
    REM = N - NS * A
    TH0 = ((T + 1) // 2 + 7) // 8 * 8
    TH1 = T - TH0

    mesh = plsc.VectorSubcoreMesh(core_axis_name="core", subcore_axis_name="subcore")

    cp = pltpu.CompilerParams()
    if "needs_layout_passes" in pltpu.CompilerParams.__dataclass_fields__:
        cp = dataclasses.replace(cp, needs_layout_passes=False)
    if "use_tc_tiling_on_sc" in pltpu.CompilerParams.__dataclass_fields__:
        cp = dataclasses.replace(cp, use_tc_tiling_on_sc=False)

    @functools.partial(
        pl.kernel,
        mesh=mesh,
        compiler_params=cp,
        out_type=[
            jax.ShapeDtypeStruct((N, DH), jnp.float32),
            jax.ShapeDtypeStruct((N, DH), jnp.float32),
            jax.ShapeDtypeStruct((N,), jnp.float32),
        ],
        scratch_types=[
            pltpu.VMEM((3, BATCH, K), jnp.int32),
            pltpu.VMEM((3, BATCH, K), jnp.int32),
            pltpu.VMEM((W,), jnp.float32),
            pltpu.VMEM((K, DH), jnp.float32),
            pltpu.VMEM((K, DH), jnp.float32),
            pltpu.VMEM((K, DH), jnp.float32),
            pltpu.VMEM((K, DH), jnp.float32),
            pltpu.VMEM((K, DH), jnp.float32),
            pltpu.VMEM((N,), jnp.float32),
            pltpu.VMEM((TH0,), jnp.int32),
            pltpu.VMEM_SHARED((N, DH), jnp.float32),
            pltpu.VMEM_SHARED((N, DH), jnp.float32),
            pltpu.SemaphoreType.DMA,
            pltpu.SemaphoreType.DMA,
            pltpu.SemaphoreType.DMA,
            pltpu.SemaphoreType.DMA,
            pltpu.SemaphoreType.DMA,
            pltpu.SemaphoreType.DMA,
            pltpu.SemaphoreType.DMA,
            pltpu.SemaphoreType.DMA,
            pltpu.SemaphoreType.DMA,
            pltpu.SemaphoreType.DMA,
            pltpu.SemaphoreType.DMA,
            pltpu.SemaphoreType.DMA,
        ],
    )
    def k(xl_hbm, xr_hbm, w_hbm, ipk_hbm, tgt_hbm,
          p0_hbm, p1_hbm, mask_hbm,
          ib0, ib1, w_v, r0, r1, r2, r3, r4, mask_v, tgt_v,
          xs, acc,
          g0, g1, g2, g3, g4, s0, s1, s2, s3, s4, b0, b1):
        c = lax.axis_index("core")
        s = lax.axis_index("subcore")
        ibufs = (ib0, ib1)
        bufs = (r0, r1, r2, r3, r4)
        gsems = (g0, g1, g2, g3, g4)
        ssems = (s0, s1, s2, s3, s4)
        bsems = (b0, b1)

        pltpu.sync_copy(w_hbm, w_v)

        @pl.loop(0, K)
        def _(r):
            for j in range(DH // L):
                r0[r, pl.ds(j * L, L)] = jnp.zeros((L,), jnp.float32)

        base = pl.multiple_of(s * A, 8)
        rows_sl = pl.ds(base, A)
        rem_base = pl.multiple_of(NS * A + s * 8, 8)
        rem_sl = pl.ds(rem_base, 8)

        for j in range(A // K):
            pltpu.sync_copy(r0, acc.at[pl.ds(base + j * K, K)])
        if A % K:
            pltpu.sync_copy(r0.at[pl.ds(0, A % K)],
                            acc.at[pl.ds(base + (A // K) * K, A % K)])

        @pl.when(s < REM // 8)
        def _():
            pltpu.sync_copy(r0.at[pl.ds(0, 8)], acc.at[rem_sl])

        @pl.when(c == 0)
        def _():
            pltpu.sync_copy(xl_hbm.at[rows_sl], xs.at[rows_sl])

            @pl.when(s < REM // 8)
            def _():
                pltpu.sync_copy(xl_hbm.at[rem_sl], xs.at[rem_sl])

        @pl.when(c == 1)
        def _():
            pltpu.sync_copy(xr_hbm.at[rows_sl], xs.at[rows_sl])

            @pl.when(s < REM // 8)
            def _():
                pltpu.sync_copy(xr_hbm.at[rem_sl], xs.at[rem_sl])

        plsc.subcore_barrier()

        def b_start(bi, pb):
            pltpu.async_copy(ipk_hbm.at[s, bi], ibufs[pb], bsems[pb])

        def b_wait(pb):
            pltpu.make_async_copy(ipk_hbm.at[s, 0], ibufs[pb],
                                  bsems[pb]).wait()

        def wval_precompute(pb):
            ib = ibufs[pb]

            @pl.loop(0, BATCH)
            def _(r):
                for g in range(K // L):
                    idx16 = ib[2, r, pl.ds(g * L, L)]
                    vals = plsc.load_gather(w_v, [idx16])
                    ib[2, r, pl.ds(g * L, L)] = plsc.bitcast(vals, jnp.int32)

        def g_start(b, pb, dib):
            pltpu.async_copy(xs.at[ibufs[pb].at[0, dib]], bufs[b], gsems[b])

        def g_wait(b):
            pltpu.make_async_copy(xs.at[ibufs[0].at[0, 0]], bufs[b],
                                  gsems[b]).wait()

        def s_start(b, pb, dib):
            pltpu.async_copy(bufs[b], acc.at[ibufs[pb].at[1, dib]],
                             ssems[b], add=True)

        def s_wait(b):
            pltpu.make_async_copy(bufs[b], acc.at[ibufs[0].at[1, 0]],
                                  ssems[b]).wait()

        def scale(b, pb, dib):
            buf = bufs[b]
            ib = ibufs[pb]

            @pl.loop(0, K, step=4)
            def _(e0):
                for dd in range(4):
                    e = e0 + dd
                    bits = plsc.load_gather(
                        ib, [jnp.full((L,), 2, jnp.int32),
                             jnp.full((L,), dib, jnp.int32),
                             jnp.full((L,), e, jnp.int32)])
                    bc = plsc.bitcast(bits, jnp.float32)
                    for j in range(DH // L):
                        sl = pl.ds(j * L, L)
                        buf[e, sl] = buf[e, sl] * bc

        def chunk(kk, bi_next, b, pb, dib, in_main):
            if dib == 0:
                wval_precompute(pb)
            if in_main and dib == 4:
                b_start(bi_next, 1 - pb)
            if in_main and dib == BATCH - 1:
                b_wait(1 - pb)

            if in_main:
                @pl.when(kk >= NRB - 1)
                def _():
                    s_wait((b + 1) % NRB)
            else:
                s_wait((b + 1) % NRB)

            nb = (b + 1) % NRB
            npb = pb if dib < BATCH - 1 else 1 - pb
            ndib = (dib + 1) % BATCH
            if in_main:
                g_start(nb, npb, ndib)
            g_wait(b)
            scale(b, pb, dib)
            s_start(b, pb, dib)

        b_start(0, 0)
        b_wait(0)
        g_start(0, 0, 0)

        @pl.loop(0, (NB - 1) // 2)
        def _(oi):
            for d20 in range(2 * BATCH):
                kk = 2 * BATCH * oi + d20
                chunk(kk, 2 * oi + d20 // BATCH + 1,
                      d20 % NRB, d20 // BATCH, d20 % BATCH, True)

        for kk in range(CH - BATCH, CH):
            b = kk % NRB
            pb = (kk // BATCH) % 2
            dib = kk % BATCH
            if dib == 0:
                wval_precompute(pb)
            s_wait((b + 1) % NRB)
            if kk + 1 < CH:
                g_start((b + 1) % NRB, pb, dib + 1)
            g_wait(b)
            scale(b, pb, dib)
            s_start(b, pb, dib)

        for j in range(CH - (NRB - 1), CH):
            s_wait(j % NRB)

        plsc.subcore_barrier()

        @pl.when(c == 0)
        def _():
            pltpu.sync_copy(acc.at[rows_sl], p0_hbm.at[rows_sl])

            @pl.when(s < REM // 8)
            def _():
                pltpu.sync_copy(acc.at[rem_sl], p0_hbm.at[rem_sl])

        @pl.when(c == 1)
        def _():
            pltpu.sync_copy(acc.at[rows_sl], p1_hbm.at[rows_sl])

            @pl.when(s < REM // 8)
            def _():
                pltpu.sync_copy(acc.at[rem_sl], p1_hbm.at[rem_sl])

        @pl.when(jnp.logical_and(c == 0, s == 0))
        def _():
            @pl.loop(0, N // L)
            def _(r):
                mask_v[pl.ds(r * L, L)] = jnp.ones((L,), jnp.float32)

            zeros16 = jnp.zeros((L,), jnp.float32)
            iota16 = lax.iota(jnp.int32, L)

            for toff, tlen in ((0, TH0), (TH0, TH1)):
                pltpu.sync_copy(tgt_hbm.at[pl.ds(toff, tlen)],
                                tgt_v.at[pl.ds(0, tlen)])
                ngroups = (tlen + L - 1) // L

                @pl.loop(0, ngroups)
                def _(g, tlen=tlen):
                    tgt16 = tgt_v[pl.ds(g * L, L)]
                    lanemask = (g * L + iota16) < tlen
                    plsc.store_scatter(mask_v, [tgt16], zeros16,
                                       mask=lanemask)

            pltpu.sync_copy(mask_v, mask_hbm)

    return k


def _combine_body(x_ref, m_ref, p0_ref, p1_ref, o_ref):
    agg = jnp.concatenate([p0_ref[...], p1_ref[...]], axis=1)
    o_ref[...] = x_ref[...] * m_ref[...] + agg


def kernel(layer_input, weights, u, v, weight_idx, targets):
    N, D = layer_input.shape
    (E,) = u.shape
    (W,) = weights.shape
    (T,) = targets.shape
    DH = D // 2
    EPT = E // NS
    NB = (EPT // K) // BATCH

    xl = layer_input[:, :DH]
    xr = layer_input[:, DH:]
    u = u.astype(jnp.int32).reshape(NS, NB, BATCH, K)
    v = v.astype(jnp.int32).reshape(NS, NB, BATCH, K)
    weight_idx = weight_idx.astype(jnp.int32).reshape(NS, NB, BATCH, K)
    ipk = jnp.stack([u, v, weight_idx], axis=2)
    targets = targets.astype(jnp.int32)

    p0, p1, mask = _sc_scatter(N, D, E, W, T)(
        xl, xr, weights, ipk, targets)

    BN = 2000
    out = pl.pallas_call(
        _combine_body,
        grid=(N // BN,),
        in_specs=[
            pl.BlockSpec((BN, D), lambda i: (i, 0)),
            pl.BlockSpec((BN, 1), lambda i: (i, 0)),
            pl.BlockSpec((BN, DH), lambda i: (i, 0)),
            pl.BlockSpec((BN, DH), lambda i: (i, 0)),
        ],
        out_specs=pl.BlockSpec((BN, D), lambda i: (i, 0)),
        out_shape=jax.ShapeDtypeStruct((N, D), jnp.float32),
    )(layer_input, mask.reshape(N, 1), p0, p1)
    return out
